# baseline (speedup 1.0000x reference)
import functools

import jax
import jax.numpy as jnp
import numpy as np
from jax import lax
from jax.experimental import pallas as pl
from jax.experimental.pallas import tpu as pltpu
from jax.experimental.pallas import tpu_sc as plsc

S = 2048
DIM = 2048
H = 16
D = 128
ROPE = 64
QLORA = 1536
TOPK = 1024
SOFTMAX_SCALE = D ** (-0.5)

BS = 256
BT = 512


def _hadamard(n):
    m = np.array([[1.0]], dtype=np.float32)
    while m.shape[0] < n:
        m = np.block([[m, m], [m, -m]]).astype(np.float32)
    return m

_HAD_NP = _hadamard(D)


def _row_mean(t):
    acc = t[:, 0:8]
    for i in range(1, 16):
        acc = acc + t[:, 8 * i:8 * i + 8]
    while acc.shape[1] > 1:
        h = acc.shape[1] // 2
        acc = acc[:, :h] + acc[:, h:]
    return acc * (1.0 / 128.0)


def _prep_kernel(x_ref, qr_ref, cos_ref, sin_ref, wq_ref, wk_ref, wp_ref,
                 lnw_ref, lnb_ref, had_ref, qf_ref, kdeq_ref, w0_ref):
    xf = x_ref[0].astype(jnp.float32)
    qrf = qr_ref[0].astype(jnp.float32)
    cosb = cos_ref[...]
    sinb = sin_ref[...]
    had = had_ref[...]

    k = jnp.dot(xf, wk_ref[...], preferred_element_type=jnp.float32)
    mu = _row_mean(k)
    var = _row_mean((k - mu) ** 2)
    k = (k - mu) / jnp.sqrt(var + 1e-6) * lnw_ref[...] + lnb_ref[...]
    k_pe = k[:, :ROPE]
    k_nope = k[:, ROPE:]
    k_rot = jnp.concatenate([-k_pe[:, ROPE // 2:], k_pe[:, :ROPE // 2]], axis=1)
    k_pe = k_pe * cosb + k_rot * sinb
    kf = jnp.dot(jnp.concatenate([k_pe, k_nope], axis=1), had,
                 preferred_element_type=jnp.float32) * (D ** -0.5)
    scale = jnp.max(jnp.abs(kf), axis=-1, keepdims=True) / 448.0 + 1e-12
    kdeq_ref[...] = (kf / scale) * scale

    w0_ref[...] = jnp.dot(xf, wp_ref[...],
                          preferred_element_type=jnp.float32) * (H ** -0.5)

    for h in range(H):
        qh = jnp.dot(qrf, wq_ref[h], preferred_element_type=jnp.float32)
        q_pe = qh[:, :ROPE]
        q_nope = qh[:, ROPE:]
        q_rot = jnp.concatenate([-q_pe[:, ROPE // 2:], q_pe[:, :ROPE // 2]],
                                axis=1)
        q_pe = q_pe * cosb + q_rot * sinb
        qf_ref[h] = jnp.dot(jnp.concatenate([q_pe, q_nope], axis=1), had,
                            preferred_element_type=jnp.float32) * (D ** -0.5)


def _score_kernel(qf_ref, kdeq_ref, w0_ref, out_ref):
    i = pl.program_id(0)
    j = pl.program_id(1)
    w0b = w0_ref[...].astype(jnp.bfloat16).astype(jnp.float32)
    kd = kdeq_ref[...]
    terms = []
    for h in range(H):
        logit = jax.lax.dot_general(
            qf_ref[h], kd, (((1,), (1,)), ((), ())),
            preferred_element_type=jnp.float32) * SOFTMAX_SCALE
        lg = jnp.maximum(logit, 0.0).astype(jnp.bfloat16).astype(jnp.float32)
        terms.append(lg * w0b[:, h][:, None])
    while len(terms) > 1:
        terms = [terms[t] + terms[t + 1] for t in range(0, len(terms), 2)]
    acc = terms[0]
    rows = i * BS + jax.lax.broadcasted_iota(jnp.int32, (BS, BT), 0)
    cols = j * BT + jax.lax.broadcasted_iota(jnp.int32, (BS, BT), 1)
    out_ref[...] = jnp.where(cols <= rows, acc, -jnp.inf)


NW = 32
R_SC = 1280
RPW = R_SC // NW
NEG_INF = jnp.float32(-jnp.inf)
MINT = jnp.int32(-2147483648)
M1 = jnp.int32(-1)
ONES = None


def _sc_topk_body(scores_hbm, vals_hbm, idx_hbm, bufA, bufB, ka, ia, kb, ib,
                  hist, vstA, vstB, istA, istB, semInA, semInB, semOA, semOB):
    wid = lax.axis_index("s") * 2 + lax.axis_index("c")
    lane = lax.iota(jnp.int32, 16)
    ones = jnp.full((16,), 1, jnp.int32)

    def sort_one(s, buf, vst, ist, semO, not_first):
        n = s + 1
        nv = (n + 15) // 16

        def enc(i, c):
            pos = i * 16 + lane
            v = buf[pl.ds(i * 16, 16)]
            b = plsc.bitcast(v, jnp.int32)
            sgn = lax.shift_right_arithmetic(b, 31)
            key = b ^ (sgn | MINT) ^ M1
            key = jnp.where(pos < n, key, M1)
            ka[pl.ds(i * 16, 16)] = key
            ia[pl.ds(i * 16, 16)] = pos
            return c
        lax.fori_loop(0, nv, enc, 0)

        for p in range(6):
            shift = 6 * p
            src_k, src_i = (ka, ia) if p % 2 == 0 else (kb, ib)
            dst_k, dst_i = (kb, ib) if p % 2 == 0 else (ka, ia)

            for hz in range(64):
                hist[pl.ds(hz * 16, 16)] = jnp.zeros((16,), jnp.int32)

            def hist_fn(i, c):
                k = plsc.load_gather(src_k, [lane * nv + i])
                d = lax.shift_right_logical(k, shift) & 63
                plsc.addupdate_scatter(hist, [d * 16 + lane], ones)
                return c
            lax.fori_loop(0, nv, hist_fn, 0)

            def scan_fn(hv, cin):
                v = hist[pl.ds(hv * 16, 16)]
                tot = jnp.sum(v)
                inc = plsc.cumsum(v)
                hist[pl.ds(hv * 16, 16)] = inc - v + cin
                return cin + tot
            lax.fori_loop(0, 64, scan_fn, jnp.int32(0))

            def perm_fn(i, c):
                gpos = lane * nv + i
                k = plsc.load_gather(src_k, [gpos])
                iv = plsc.load_gather(src_i, [gpos])
                d = lax.shift_right_logical(k, shift) & 63
                bin_ = d * 16 + lane
                off = plsc.load_gather(hist, [bin_])
                plsc.store_scatter(dst_k, [off], k)
                plsc.store_scatter(dst_i, [off], iv)
                plsc.addupdate_scatter(hist, [bin_], ones)
                return c
            lax.fori_loop(0, nv, perm_fn, 0)

        @pl.when(not_first)
        def _():
            pltpu.make_async_copy(vst, vals_hbm.at[s], semO).wait()
            pltpu.make_async_copy(ist, idx_hbm.at[s], semO).wait()

        def out_fn(j, c):
            rank = j * 16 + lane
            key = ka[pl.ds(j * 16, 16)]
            iv = ia[pl.ds(j * 16, 16)]
            asc = key ^ M1
            b = jnp.where(asc < 0, asc ^ MINT, asc ^ M1)
            val = plsc.bitcast(b, jnp.float32)
            valid = rank < n
            vst[pl.ds(j * 16, 16)] = jnp.where(valid, val, NEG_INF)
            ist[pl.ds(j * 16, 16)] = jnp.where(valid, iv, M1)
            return c
        lax.fori_loop(0, TOPK // 16, out_fn, 0)

        pltpu.async_copy(vst, vals_hbm.at[s], semO)
        pltpu.async_copy(ist, idx_hbm.at[s], semO)

    pltpu.async_copy(scores_hbm.at[wid], bufA, semInA)

    def pair(q, carry):
        s0 = (2 * q) * NW + wid
        pltpu.make_async_copy(scores_hbm.at[s0], bufA, semInA).wait()
        pltpu.async_copy(scores_hbm.at[s0 + NW], bufB, semInB)
        sort_one(s0, bufA, vstA, istA, semOA, q > 0)
        pltpu.make_async_copy(scores_hbm.at[s0 + NW], bufB, semInB).wait()

        @pl.when(q < RPW // 2 - 1)
        def _():
            pltpu.async_copy(scores_hbm.at[s0 + 2 * NW], bufA, semInA)
        sort_one(s0 + NW, bufB, vstB, istB, semOB, q > 0)
        return carry

    lax.fori_loop(0, RPW // 2, pair, 0)

    last_a = (RPW - 2) * NW + wid
    last_b = (RPW - 1) * NW + wid
    pltpu.make_async_copy(vstA, vals_hbm.at[last_a], semOA).wait()
    pltpu.make_async_copy(istA, idx_hbm.at[last_a], semOA).wait()
    pltpu.make_async_copy(vstB, vals_hbm.at[last_b], semOB).wait()
    pltpu.make_async_copy(istB, idx_hbm.at[last_b], semOB).wait()


@functools.partial(
    pl.kernel,
    mesh=plsc.VectorSubcoreMesh(core_axis_name="c", subcore_axis_name="s"),
    compiler_params=pltpu.CompilerParams(needs_layout_passes=False),
    out_type=[jax.ShapeDtypeStruct((R_SC, TOPK), jnp.float32),
              jax.ShapeDtypeStruct((R_SC, TOPK), jnp.int32)],
    scratch_types=[pltpu.VMEM((S,), jnp.float32),
                   pltpu.VMEM((S,), jnp.float32),
                   pltpu.VMEM((S,), jnp.int32),
                   pltpu.VMEM((S,), jnp.int32),
                   pltpu.VMEM((S,), jnp.int32),
                   pltpu.VMEM((S,), jnp.int32),
                   pltpu.VMEM((1024,), jnp.int32),
                   pltpu.VMEM((TOPK,), jnp.float32),
                   pltpu.VMEM((TOPK,), jnp.float32),
                   pltpu.VMEM((TOPK,), jnp.int32),
                   pltpu.VMEM((TOPK,), jnp.int32),
                   pltpu.SemaphoreType.DMA,
                   pltpu.SemaphoreType.DMA,
                   pltpu.SemaphoreType.DMA,
                   pltpu.SemaphoreType.DMA],
)
def sc_topk(scores_hbm, vals_hbm, idx_hbm, bufA, bufB, ka, ia, kb, ib, hist,
            vstA, vstB, istA, istB, semInA, semInB, semOA, semOB):
    _sc_topk_body(scores_hbm, vals_hbm, idx_hbm, bufA, bufB, ka, ia, kb, ib,
                  hist, vstA, vstB, istA, istB, semInA, semInB, semOA, semOB)


def kernel(x, qr, cos, sin, k_cache, k_scale, Wq, Wk, ln_w, ln_b, Wp):
    del k_cache, k_scale
    had = jnp.asarray(_HAD_NP)
    wq_h = Wq.reshape(QLORA, H, D).transpose(1, 0, 2)

    nb = S // BS
    qf, kdeq, w0 = pl.pallas_call(
        _prep_kernel,
        grid=(nb,),
        in_specs=[
            pl.BlockSpec((1, BS, DIM), lambda i: (0, i, 0)),
            pl.BlockSpec((1, BS, QLORA), lambda i: (0, i, 0)),
            pl.BlockSpec((BS, ROPE), lambda i: (i, 0)),
            pl.BlockSpec((BS, ROPE), lambda i: (i, 0)),
            pl.BlockSpec((H, QLORA, D), lambda i: (0, 0, 0)),
            pl.BlockSpec((DIM, D), lambda i: (0, 0)),
            pl.BlockSpec((DIM, H), lambda i: (0, 0)),
            pl.BlockSpec((D,), lambda i: (0,)),
            pl.BlockSpec((D,), lambda i: (0,)),
            pl.BlockSpec((D, D), lambda i: (0, 0)),
        ],
        out_specs=[
            pl.BlockSpec((H, BS, D), lambda i: (0, i, 0)),
            pl.BlockSpec((BS, D), lambda i: (i, 0)),
            pl.BlockSpec((BS, H), lambda i: (i, 0)),
        ],
        out_shape=[
            jax.ShapeDtypeStruct((H, S, D), jnp.float32),
            jax.ShapeDtypeStruct((S, D), jnp.float32),
            jax.ShapeDtypeStruct((S, H), jnp.float32),
        ],
    )(x, qr, cos, sin, wq_h, Wk, Wp, ln_w, ln_b, had)

    scores = pl.pallas_call(
        _score_kernel,
        grid=(S // BS, S // BT),
        in_specs=[
            pl.BlockSpec((H, BS, D), lambda i, j: (0, i, 0)),
            pl.BlockSpec((BT, D), lambda i, j: (j, 0)),
            pl.BlockSpec((BS, H), lambda i, j: (i, 0)),
        ],
        out_specs=pl.BlockSpec((BS, BT), lambda i, j: (i, j)),
        out_shape=jax.ShapeDtypeStruct((S, S), jnp.float32),
    )(qf, kdeq, w0)

    vals1, idx1 = sc_topk(scores)
    vals2, idx2 = jax.lax.top_k(scores[R_SC:], TOPK)
    idx2 = jnp.where(jnp.isinf(vals2), -1, idx2)
    return (jnp.concatenate([vals1, vals2], axis=0),
            jnp.concatenate([idx1, idx2], axis=0))


# speedup vs baseline: 1.1078x; 1.1078x over previous
import functools

import jax
import jax.numpy as jnp
import numpy as np
from jax import lax
from jax.experimental import pallas as pl
from jax.experimental.pallas import tpu as pltpu
from jax.experimental.pallas import tpu_sc as plsc

S = 2048
DIM = 2048
H = 16
D = 128
ROPE = 64
QLORA = 1536
TOPK = 1024
SOFTMAX_SCALE = D ** (-0.5)

BS = 256
BT = 512


def _hadamard(n):
    m = np.array([[1.0]], dtype=np.float32)
    while m.shape[0] < n:
        m = np.block([[m, m], [m, -m]]).astype(np.float32)
    return m

_HAD_NP = _hadamard(D)


def _row_mean(t):
    acc = t[:, 0:8]
    for i in range(1, 16):
        acc = acc + t[:, 8 * i:8 * i + 8]
    while acc.shape[1] > 1:
        h = acc.shape[1] // 2
        acc = acc[:, :h] + acc[:, h:]
    return acc * (1.0 / 128.0)


def _prep_kernel(x_ref, qr_ref, cos_ref, sin_ref, wq_ref, wk_ref, wp_ref,
                 lnw_ref, lnb_ref, had_ref, qf_ref, kdeq_ref, w0_ref):
    xf = x_ref[0].astype(jnp.float32)
    qrf = qr_ref[0].astype(jnp.float32)
    cosb = cos_ref[...]
    sinb = sin_ref[...]
    had = had_ref[...]

    k = jnp.dot(xf, wk_ref[...], preferred_element_type=jnp.float32)
    mu = _row_mean(k)
    var = _row_mean((k - mu) ** 2)
    k = (k - mu) / jnp.sqrt(var + 1e-6) * lnw_ref[...] + lnb_ref[...]
    k_pe = k[:, :ROPE]
    k_nope = k[:, ROPE:]
    k_rot = jnp.concatenate([-k_pe[:, ROPE // 2:], k_pe[:, :ROPE // 2]], axis=1)
    k_pe = k_pe * cosb + k_rot * sinb
    kf = jnp.dot(jnp.concatenate([k_pe, k_nope], axis=1), had,
                 preferred_element_type=jnp.float32) * (D ** -0.5)
    scale = jnp.max(jnp.abs(kf), axis=-1, keepdims=True) / 448.0 + 1e-12
    kdeq_ref[...] = (kf / scale) * scale

    w0_ref[...] = jnp.dot(xf, wp_ref[...],
                          preferred_element_type=jnp.float32) * (H ** -0.5)

    for h in range(H):
        qh = jnp.dot(qrf, wq_ref[h], preferred_element_type=jnp.float32)
        q_pe = qh[:, :ROPE]
        q_nope = qh[:, ROPE:]
        q_rot = jnp.concatenate([-q_pe[:, ROPE // 2:], q_pe[:, :ROPE // 2]],
                                axis=1)
        q_pe = q_pe * cosb + q_rot * sinb
        qf_ref[h] = jnp.dot(jnp.concatenate([q_pe, q_nope], axis=1), had,
                            preferred_element_type=jnp.float32) * (D ** -0.5)


def _score_kernel(qf_ref, kdeq_ref, w0_ref, out_ref):
    i = pl.program_id(0)
    j = pl.program_id(1)
    w0b = w0_ref[...].astype(jnp.bfloat16).astype(jnp.float32)
    kd = kdeq_ref[...]
    terms = []
    for h in range(H):
        logit = jax.lax.dot_general(
            qf_ref[h], kd, (((1,), (1,)), ((), ())),
            preferred_element_type=jnp.float32) * SOFTMAX_SCALE
        lg = jnp.maximum(logit, 0.0).astype(jnp.bfloat16).astype(jnp.float32)
        terms.append(lg * w0b[:, h][:, None])
    while len(terms) > 1:
        terms = [terms[t] + terms[t + 1] for t in range(0, len(terms), 2)]
    acc = terms[0]
    rows = i * BS + jax.lax.broadcasted_iota(jnp.int32, (BS, BT), 0)
    cols = j * BT + jax.lax.broadcasted_iota(jnp.int32, (BS, BT), 1)
    out_ref[...] = jnp.where(cols <= rows, acc, -jnp.inf)


NW = 32
R_SC = 1152
RPW = R_SC // NW
NEG_INF = jnp.float32(-jnp.inf)
MINT = jnp.int32(-2147483648)
M1 = jnp.int32(-1)
ONES = None


def _sc_topk_body(scores_hbm, vals_hbm, idx_hbm, bufA, bufB, ka, ia, kb, ib,
                  hist, vstA, vstB, istA, istB, semInA, semInB, semOA, semOB):
    wid = lax.axis_index("s") * 2 + lax.axis_index("c")
    lane = lax.iota(jnp.int32, 16)
    ones = jnp.full((16,), 1, jnp.int32)

    def sort_one(s, buf, vst, ist, semO, not_first):
        n = s + 1
        nv = (n + 15) // 16

        def enc(i, c):
            pos = i * 16 + lane
            v = buf[pl.ds(i * 16, 16)]
            b = plsc.bitcast(v, jnp.int32)
            sgn = lax.shift_right_arithmetic(b, 31)
            key = b ^ (sgn | MINT) ^ M1
            key = jnp.where(pos < n, key, M1)
            ka[pl.ds(i * 16, 16)] = key
            ia[pl.ds(i * 16, 16)] = pos
            return c
        lax.fori_loop(0, nv, enc, 0)

        for p in range(6):
            shift = 6 * p
            src_k, src_i = (ka, ia) if p % 2 == 0 else (kb, ib)
            dst_k, dst_i = (kb, ib) if p % 2 == 0 else (ka, ia)

            for hz in range(64):
                hist[pl.ds(hz * 16, 16)] = jnp.zeros((16,), jnp.int32)

            def hist_fn(i, c):
                k = plsc.load_gather(src_k, [lane * nv + i])
                d = lax.shift_right_logical(k, shift) & 63
                plsc.addupdate_scatter(hist, [d * 16 + lane], ones)
                return c
            lax.fori_loop(0, nv, hist_fn, 0)

            def scan_fn(hv, cin):
                v = hist[pl.ds(hv * 16, 16)]
                tot = jnp.sum(v)
                inc = plsc.cumsum(v)
                hist[pl.ds(hv * 16, 16)] = inc - v + cin
                return cin + tot
            lax.fori_loop(0, 64, scan_fn, jnp.int32(0))

            def perm_fn(i, c):
                gpos = lane * nv + i
                k = plsc.load_gather(src_k, [gpos])
                iv = plsc.load_gather(src_i, [gpos])
                d = lax.shift_right_logical(k, shift) & 63
                bin_ = d * 16 + lane
                off = plsc.load_gather(hist, [bin_])
                plsc.store_scatter(dst_k, [off], k)
                plsc.store_scatter(dst_i, [off], iv)
                plsc.addupdate_scatter(hist, [bin_], ones)
                return c
            lax.fori_loop(0, nv, perm_fn, 0)

        @pl.when(not_first)
        def _():
            pltpu.make_async_copy(vst, vals_hbm.at[s], semO).wait()
            pltpu.make_async_copy(ist, idx_hbm.at[s], semO).wait()

        def out_fn(j, c):
            rank = j * 16 + lane
            key = ka[pl.ds(j * 16, 16)]
            iv = ia[pl.ds(j * 16, 16)]
            asc = key ^ M1
            b = jnp.where(asc < 0, asc ^ MINT, asc ^ M1)
            val = plsc.bitcast(b, jnp.float32)
            valid = rank < n
            vst[pl.ds(j * 16, 16)] = jnp.where(valid, val, NEG_INF)
            ist[pl.ds(j * 16, 16)] = jnp.where(valid, iv, M1)
            return c
        lax.fori_loop(0, TOPK // 16, out_fn, 0)

        pltpu.async_copy(vst, vals_hbm.at[s], semO)
        pltpu.async_copy(ist, idx_hbm.at[s], semO)

    pltpu.async_copy(scores_hbm.at[wid], bufA, semInA)

    def pair(q, carry):
        s0 = (2 * q) * NW + wid
        pltpu.make_async_copy(scores_hbm.at[s0], bufA, semInA).wait()
        pltpu.async_copy(scores_hbm.at[s0 + NW], bufB, semInB)
        sort_one(s0, bufA, vstA, istA, semOA, q > 0)
        pltpu.make_async_copy(scores_hbm.at[s0 + NW], bufB, semInB).wait()

        @pl.when(q < RPW // 2 - 1)
        def _():
            pltpu.async_copy(scores_hbm.at[s0 + 2 * NW], bufA, semInA)
        sort_one(s0 + NW, bufB, vstB, istB, semOB, q > 0)
        return carry

    lax.fori_loop(0, RPW // 2, pair, 0)

    last_a = (RPW - 2) * NW + wid
    last_b = (RPW - 1) * NW + wid
    pltpu.make_async_copy(vstA, vals_hbm.at[last_a], semOA).wait()
    pltpu.make_async_copy(istA, idx_hbm.at[last_a], semOA).wait()
    pltpu.make_async_copy(vstB, vals_hbm.at[last_b], semOB).wait()
    pltpu.make_async_copy(istB, idx_hbm.at[last_b], semOB).wait()


@functools.partial(
    pl.kernel,
    mesh=plsc.VectorSubcoreMesh(core_axis_name="c", subcore_axis_name="s"),
    compiler_params=pltpu.CompilerParams(needs_layout_passes=False),
    out_type=[jax.ShapeDtypeStruct((R_SC, TOPK), jnp.float32),
              jax.ShapeDtypeStruct((R_SC, TOPK), jnp.int32)],
    scratch_types=[pltpu.VMEM((S,), jnp.float32),
                   pltpu.VMEM((S,), jnp.float32),
                   pltpu.VMEM((S,), jnp.int32),
                   pltpu.VMEM((S,), jnp.int32),
                   pltpu.VMEM((S,), jnp.int32),
                   pltpu.VMEM((S,), jnp.int32),
                   pltpu.VMEM((1024,), jnp.int32),
                   pltpu.VMEM((TOPK,), jnp.float32),
                   pltpu.VMEM((TOPK,), jnp.float32),
                   pltpu.VMEM((TOPK,), jnp.int32),
                   pltpu.VMEM((TOPK,), jnp.int32),
                   pltpu.SemaphoreType.DMA,
                   pltpu.SemaphoreType.DMA,
                   pltpu.SemaphoreType.DMA,
                   pltpu.SemaphoreType.DMA],
)
def sc_topk(scores_hbm, vals_hbm, idx_hbm, bufA, bufB, ka, ia, kb, ib, hist,
            vstA, vstB, istA, istB, semInA, semInB, semOA, semOB):
    _sc_topk_body(scores_hbm, vals_hbm, idx_hbm, bufA, bufB, ka, ia, kb, ib,
                  hist, vstA, vstB, istA, istB, semInA, semInB, semOA, semOB)


def kernel(x, qr, cos, sin, k_cache, k_scale, Wq, Wk, ln_w, ln_b, Wp):
    del k_cache, k_scale
    had = jnp.asarray(_HAD_NP)
    wq_h = Wq.reshape(QLORA, H, D).transpose(1, 0, 2)

    nb = S // BS
    qf, kdeq, w0 = pl.pallas_call(
        _prep_kernel,
        grid=(nb,),
        in_specs=[
            pl.BlockSpec((1, BS, DIM), lambda i: (0, i, 0)),
            pl.BlockSpec((1, BS, QLORA), lambda i: (0, i, 0)),
            pl.BlockSpec((BS, ROPE), lambda i: (i, 0)),
            pl.BlockSpec((BS, ROPE), lambda i: (i, 0)),
            pl.BlockSpec((H, QLORA, D), lambda i: (0, 0, 0)),
            pl.BlockSpec((DIM, D), lambda i: (0, 0)),
            pl.BlockSpec((DIM, H), lambda i: (0, 0)),
            pl.BlockSpec((D,), lambda i: (0,)),
            pl.BlockSpec((D,), lambda i: (0,)),
            pl.BlockSpec((D, D), lambda i: (0, 0)),
        ],
        out_specs=[
            pl.BlockSpec((H, BS, D), lambda i: (0, i, 0)),
            pl.BlockSpec((BS, D), lambda i: (i, 0)),
            pl.BlockSpec((BS, H), lambda i: (i, 0)),
        ],
        out_shape=[
            jax.ShapeDtypeStruct((H, S, D), jnp.float32),
            jax.ShapeDtypeStruct((S, D), jnp.float32),
            jax.ShapeDtypeStruct((S, H), jnp.float32),
        ],
    )(x, qr, cos, sin, wq_h, Wk, Wp, ln_w, ln_b, had)

    scores = pl.pallas_call(
        _score_kernel,
        grid=(S // BS, S // BT),
        in_specs=[
            pl.BlockSpec((H, BS, D), lambda i, j: (0, i, 0)),
            pl.BlockSpec((BT, D), lambda i, j: (j, 0)),
            pl.BlockSpec((BS, H), lambda i, j: (i, 0)),
        ],
        out_specs=pl.BlockSpec((BS, BT), lambda i, j: (i, j)),
        out_shape=jax.ShapeDtypeStruct((S, S), jnp.float32),
    )(qf, kdeq, w0)

    vals1, idx1 = sc_topk(scores)
    vals2, idx2 = jax.lax.top_k(scores[R_SC:], TOPK)
    idx2 = jnp.where(jnp.isinf(vals2), -1, idx2)
    return (jnp.concatenate([vals1, vals2], axis=0),
            jnp.concatenate([idx1, idx2], axis=0))
